# baseline (speedup 1.0000x reference)
import jax
import jax.numpy as jnp
from jax import lax
from jax.experimental import pallas as pl
from jax.experimental.pallas import tpu as pltpu
from jax.experimental.pallas import tpu_sc as plsc

B = 16384
D = 64
NC = 2
NS = 16
NW = NC * NS
BPW = B // NW
RB = 128
NB = BPW // RB
GPB = RB // 16
L = 16


def _rsqrt(x):
    i = plsc.bitcast(x, jnp.int32)
    y = plsc.bitcast(jnp.int32(0x5F3759DF) - (i >> 1), jnp.float32)
    for _ in range(3):
        y = y * (1.5 - 0.5 * x * y * y)
    return y


def _body(hidx_hbm, ridx_hbm, tidx_hbm, ent_hbm, rel_hbm, out_hbm,
          hidx_v, ridx_v, tidx_v,
          hrow_a, trow_a, rrow_a, hrow_b, trow_b, rrow_b, out_v,
          idx_sem, sem_a, sem_b):
    wid = lax.axis_index("s") * NC + lax.axis_index("c")
    base = wid * BPW

    pltpu.async_copy(hidx_hbm.at[pl.ds(base, BPW)], hidx_v, idx_sem)
    pltpu.async_copy(ridx_hbm.at[pl.ds(base, BPW)], ridx_v, idx_sem)
    cp = pltpu.async_copy(tidx_hbm.at[pl.ds(base, BPW)], tidx_v, idx_sem)
    cp.wait()
    cp.wait()
    cp.wait()

    lane = lax.iota(jnp.int32, L)
    lane4 = lane * 8

    def issue(b, hrow_v, trow_v, rrow_v, sem):
        def one(g, c2):
            gb = b * RB + g * L
            hv = hidx_v[pl.ds(gb, L)]
            rv = ridx_v[pl.ds(gb, L)]
            tv = tidx_v[pl.ds(gb, L)]
            for j in range(L):
                dr = g * L + j
                pltpu.async_copy(ent_hbm.at[pl.ds(hv[j], 1)],
                                 hrow_v.at[pl.ds(dr, 1)], sem)
                pltpu.async_copy(ent_hbm.at[pl.ds(tv[j], 1)],
                                 trow_v.at[pl.ds(dr, 1)], sem)
                pltpu.async_copy(rel_hbm.at[pl.ds(rv[j], 1)],
                                 rrow_v.at[pl.ds(dr, 1)], sem)
            return c2

        lax.fori_loop(0, GPB, one, 0)

    def drain(hrow_v, sem):
        dsc = pltpu.make_async_copy(
            ent_hbm.at[pl.ds(0, 1)], hrow_v.at[pl.ds(0, 1)], sem)

        def one(i, c2):
            dsc.wait()
            return c2

        lax.fori_loop(0, 3 * RB, one, 0)

    def compute(b, hrow_v, trow_v, rrow_v):
        def group(g, c2):
            row = g * L + lane
            zero = jnp.zeros((L,), jnp.float32)
            nh = zero
            nt = zero
            nr = zero
            hr = zero
            ht = zero
            tr = zero
            for d in range(D):
                col = (lane4 + d) & (D - 1)
                h = plsc.load_gather(hrow_v, [row, col])
                t = plsc.load_gather(trow_v, [row, col])
                r = plsc.load_gather(rrow_v, [row, col])
                nh = nh + h * h
                nt = nt + t * t
                nr = nr + r * r
                hr = hr + h * r
                ht = ht + h * t
                tr = tr + t * r
            inh = jnp.minimum(_rsqrt(nh), 1e12)
            int_ = jnp.minimum(_rsqrt(nt), 1e12)
            s = (nh * inh * inh + nt * int_ * int_ + nr
                 + 2.0 * (hr * inh - ht * (inh * int_) - tr * int_))
            s = jnp.maximum(s, 0.0)
            out_v[pl.ds(b * RB + g * L, L)] = s * _rsqrt(s)
            return c2

        lax.fori_loop(0, GPB, group, 0)

    bufs = ((hrow_a, trow_a, rrow_a, sem_a), (hrow_b, trow_b, rrow_b, sem_b))
    issue(0, *bufs[0])
    issue(1, *bufs[1])
    for b in range(NB):
        hv, tv, rv, sem = bufs[b % 2]
        drain(hv, sem)
        compute(b, hv, tv, rv)
        if b + 2 < NB:
            issue(b + 2, *bufs[b % 2])

    pltpu.sync_copy(out_v, out_hbm.at[pl.ds(base, BPW)])


@jax.jit
def _transe(h_idx, r_idx, t_idx, entity_emb, rel_emb):
    mesh = plsc.VectorSubcoreMesh(core_axis_name="c", subcore_axis_name="s",
                                  num_cores=NC, num_subcores=NS)
    row_t = pltpu.VMEM((RB, D), jnp.float32)
    f = pl.kernel(
        _body,
        out_type=jax.ShapeDtypeStruct((B,), jnp.float32),
        mesh=mesh,
        compiler_params=pltpu.CompilerParams(needs_layout_passes=False,
                                             use_tc_tiling_on_sc=True),
        scratch_types=[
            pltpu.VMEM((BPW,), jnp.int32),
            pltpu.VMEM((BPW,), jnp.int32),
            pltpu.VMEM((BPW,), jnp.int32),
            row_t, row_t, row_t, row_t, row_t, row_t,
            pltpu.VMEM((BPW,), jnp.float32),
            pltpu.SemaphoreType.DMA,
            pltpu.SemaphoreType.DMA,
            pltpu.SemaphoreType.DMA,
        ],
    )
    return f(h_idx, r_idx, t_idx, entity_emb, rel_emb)


def kernel(h_idx, r_idx, t_idx, entity_emb, rel_emb):
    hi = h_idx.astype(jnp.int32)
    pidx = hi[:8]
    pval = jnp.take(entity_emb, pidx, axis=0)
    ent_f = entity_emb.at[pidx].set(pval)
    return _transe(hi, r_idx.astype(jnp.int32), t_idx.astype(jnp.int32),
                   ent_f, rel_emb)


# speedup vs baseline: 1.0213x; 1.0213x over previous
import jax
import jax.numpy as jnp
from jax import lax
from jax.experimental import pallas as pl
from jax.experimental.pallas import tpu as pltpu
from jax.experimental.pallas import tpu_sc as plsc

B = 16384
D = 64
NC = 2
NS = 16
NW = NC * NS
BPW = B // NW
RB = 128
NB = BPW // RB
GPB = RB // 16
L = 16


def _rsqrt(x):
    i = plsc.bitcast(x, jnp.int32)
    y = plsc.bitcast(jnp.int32(0x5F3759DF) - (i >> 1), jnp.float32)
    for _ in range(3):
        y = y * (1.5 - 0.5 * x * y * y)
    return y


def _body(hidx_hbm, ridx_hbm, tidx_hbm, ent_hbm, rel_hbm, out_hbm,
          hidx_v, ridx_v, tidx_v,
          hrow_a, trow_a, rrow_a, hrow_b, trow_b, rrow_b, out_v,
          idx_sem, sem_a, sem_b):
    wid = lax.axis_index("s") * NC + lax.axis_index("c")
    base = wid * BPW

    pltpu.async_copy(hidx_hbm.at[pl.ds(base, BPW)], hidx_v, idx_sem)
    pltpu.async_copy(ridx_hbm.at[pl.ds(base, BPW)], ridx_v, idx_sem)
    cp = pltpu.async_copy(tidx_hbm.at[pl.ds(base, BPW)], tidx_v, idx_sem)
    cp.wait()
    cp.wait()
    cp.wait()

    lane = lax.iota(jnp.int32, L)
    lane4 = lane * 1

    def issue(b, hrow_v, trow_v, rrow_v, sem):
        def one(g, c2):
            gb = b * RB + g * L
            hv = hidx_v[pl.ds(gb, L)]
            rv = ridx_v[pl.ds(gb, L)]
            tv = tidx_v[pl.ds(gb, L)]
            for j in range(L):
                dr = g * L + j
                pltpu.async_copy(ent_hbm.at[pl.ds(hv[j], 1)],
                                 hrow_v.at[pl.ds(dr, 1)], sem)
                pltpu.async_copy(ent_hbm.at[pl.ds(tv[j], 1)],
                                 trow_v.at[pl.ds(dr, 1)], sem)
                pltpu.async_copy(rel_hbm.at[pl.ds(rv[j], 1)],
                                 rrow_v.at[pl.ds(dr, 1)], sem)
            return c2

        lax.fori_loop(0, GPB, one, 0)

    def drain(hrow_v, sem):
        dsc = pltpu.make_async_copy(
            ent_hbm.at[pl.ds(0, 1)], hrow_v.at[pl.ds(0, 1)], sem)

        def one(i, c2):
            dsc.wait()
            return c2

        lax.fori_loop(0, 3 * RB, one, 0)

    def compute(b, hrow_v, trow_v, rrow_v):
        def group(g, c2):
            row = g * L + lane
            zero = jnp.zeros((L,), jnp.float32)
            nh = zero
            nt = zero
            nr = zero
            hr = zero
            ht = zero
            tr = zero
            for d in range(D):
                col = (lane4 + d) & (D - 1)
                h = plsc.load_gather(hrow_v, [row, col])
                t = plsc.load_gather(trow_v, [row, col])
                r = plsc.load_gather(rrow_v, [row, col])
                nh = nh + h * h
                nt = nt + t * t
                nr = nr + r * r
                hr = hr + h * r
                ht = ht + h * t
                tr = tr + t * r
            inh = jnp.minimum(_rsqrt(nh), 1e12)
            int_ = jnp.minimum(_rsqrt(nt), 1e12)
            s = (nh * inh * inh + nt * int_ * int_ + nr
                 + 2.0 * (hr * inh - ht * (inh * int_) - tr * int_))
            s = jnp.maximum(s, 0.0)
            out_v[pl.ds(b * RB + g * L, L)] = s * _rsqrt(s)
            return c2

        lax.fori_loop(0, GPB, group, 0)

    bufs = ((hrow_a, trow_a, rrow_a, sem_a), (hrow_b, trow_b, rrow_b, sem_b))
    issue(0, *bufs[0])
    issue(1, *bufs[1])
    for b in range(NB):
        hv, tv, rv, sem = bufs[b % 2]
        drain(hv, sem)
        compute(b, hv, tv, rv)
        if b + 2 < NB:
            issue(b + 2, *bufs[b % 2])

    pltpu.sync_copy(out_v, out_hbm.at[pl.ds(base, BPW)])


@jax.jit
def _transe(h_idx, r_idx, t_idx, entity_emb, rel_emb):
    mesh = plsc.VectorSubcoreMesh(core_axis_name="c", subcore_axis_name="s",
                                  num_cores=NC, num_subcores=NS)
    row_t = pltpu.VMEM((RB, D), jnp.float32)
    f = pl.kernel(
        _body,
        out_type=jax.ShapeDtypeStruct((B,), jnp.float32),
        mesh=mesh,
        compiler_params=pltpu.CompilerParams(needs_layout_passes=False,
                                             use_tc_tiling_on_sc=True),
        scratch_types=[
            pltpu.VMEM((BPW,), jnp.int32),
            pltpu.VMEM((BPW,), jnp.int32),
            pltpu.VMEM((BPW,), jnp.int32),
            row_t, row_t, row_t, row_t, row_t, row_t,
            pltpu.VMEM((BPW,), jnp.float32),
            pltpu.SemaphoreType.DMA,
            pltpu.SemaphoreType.DMA,
            pltpu.SemaphoreType.DMA,
        ],
    )
    return f(h_idx, r_idx, t_idx, entity_emb, rel_emb)


def kernel(h_idx, r_idx, t_idx, entity_emb, rel_emb):
    hi = h_idx.astype(jnp.int32)
    pidx = hi[:8]
    pval = jnp.take(entity_emb, pidx, axis=0)
    ent_f = entity_emb.at[pidx].set(pval)
    return _transe(hi, r_idx.astype(jnp.int32), t_idx.astype(jnp.int32),
                   ent_f, rel_emb)
